# baseline (speedup 1.0000x reference)
import jax
import jax.numpy as jnp
from jax.experimental import pallas as pl

B, N, T, H = 8, 325, 168, 64
N_BLK = 65
LN = 128
SL = T * H // LN


def _gather_body(hour_ref, start_ref, end_ref, pos_ref, out_ref):
    row = jax.lax.broadcasted_iota(jnp.int32, (24, 1), 0)
    table = (pos_ref[...]
             + jnp.where(row == 0, 1.0, 0.0) * start_ref[...][None, :]
             + jnp.where(row == 23, 1.0, 0.0) * end_ref[...][None, :])
    hour = hour_ref[0, 0, :]
    col = jax.lax.broadcasted_iota(jnp.int32, (T, 24), 1)
    onehot = (hour[:, None] == col).astype(jnp.float32)
    out_ref[0] = jnp.dot(onehot, table, preferred_element_type=jnp.float32)


def _add_body(node_ref, add_ref, out_ref):
    out_ref[...] = node_ref[...] + add_ref[...]


def kernel(node_emb, hour_of_day, session_start, session_end, position_emb):
    hour3 = hour_of_day.astype(jnp.int32).reshape(B, 1, T)
    add = pl.pallas_call(
        _gather_body,
        grid=(B,),
        in_specs=[
            pl.BlockSpec((1, 1, T), lambda b: (b, 0, 0)),
            pl.BlockSpec((H,), lambda b: (0,)),
            pl.BlockSpec((H,), lambda b: (0,)),
            pl.BlockSpec((24, H), lambda b: (0, 0)),
        ],
        out_specs=pl.BlockSpec((1, T, H), lambda b: (b, 0, 0)),
        out_shape=jax.ShapeDtypeStruct((B, T, H), jnp.float32),
    )(hour3, session_start, session_end, position_emb)

    node2 = node_emb.reshape(B, N, SL, LN)
    add2 = add.reshape(B, 1, SL, LN)
    out2 = pl.pallas_call(
        _add_body,
        grid=(B, N // N_BLK),
        in_specs=[
            pl.BlockSpec((1, N_BLK, SL, LN), lambda b, n: (b, n, 0, 0)),
            pl.BlockSpec((1, 1, SL, LN), lambda b, n: (b, 0, 0, 0)),
        ],
        out_specs=pl.BlockSpec((1, N_BLK, SL, LN), lambda b, n: (b, n, 0, 0)),
        out_shape=jax.ShapeDtypeStruct((B, N, SL, LN), jnp.float32),
    )(node2, add2)
    return out2.reshape(B, N, T, H)


# speedup vs baseline: 1.0148x; 1.0148x over previous
import jax
import jax.numpy as jnp
from jax.experimental import pallas as pl
from jax.experimental.pallas import tpu as pltpu

B, N, T, H = 8, 325, 168, 64
LN = 128
SL = T * H // LN
N_BLK = 25
CHUNKS = N // N_BLK
NC = B * CHUNKS
Q = 4
G = NC // Q


def _gather_body(hour_ref, start_ref, end_ref, pos_ref, out_ref):
    row = jax.lax.broadcasted_iota(jnp.int32, (24, 1), 0)
    table = (pos_ref[...]
             + jnp.where(row == 0, 1.0, 0.0) * start_ref[...][None, :]
             + jnp.where(row == 23, 1.0, 0.0) * end_ref[...][None, :])
    hour = hour_ref[0, 0, :]
    col = jax.lax.broadcasted_iota(jnp.int32, (T, 24), 1)
    onehot = (hour[:, None] == col).astype(jnp.float32)
    out_ref[0] = jnp.dot(onehot, table, preferred_element_type=jnp.float32)


def _stream_body(add_ref, node_ref, out_ref, ibuf, obuf, isem, osem):
    def in_copy(i, slot):
        b = i // CHUNKS
        c = jax.lax.rem(i, CHUNKS)
        return pltpu.make_async_copy(
            node_ref.at[b, pl.ds(c * N_BLK, N_BLK)], ibuf.at[slot],
            isem.at[slot])

    def out_copy(i, slot):
        b = i // CHUNKS
        c = jax.lax.rem(i, CHUNKS)
        return pltpu.make_async_copy(
            obuf.at[slot], out_ref.at[b, pl.ds(c * N_BLK, N_BLK)],
            osem.at[slot])

    for j in range(Q):
        in_copy(j, j).start()

    def group(g, carry):
        for j in range(Q):
            i = g * Q + j
            in_copy(i, j).wait()

            @pl.when(g > 0)
            def _():
                out_copy(i - Q, j).wait()

            b = i // CHUNKS
            obuf[j] = ibuf[j] + add_ref[b][None, :, :]
            out_copy(i, j).start()

            @pl.when(g < G - 1)
            def _():
                in_copy(i + Q, j).start()
        return carry

    jax.lax.fori_loop(0, G, group, 0)
    for j in range(Q):
        out_copy(NC - Q + j, j).wait()


def kernel(node_emb, hour_of_day, session_start, session_end, position_emb):
    hour3 = hour_of_day.astype(jnp.int32).reshape(B, 1, T)
    add = pl.pallas_call(
        _gather_body,
        grid=(B,),
        in_specs=[
            pl.BlockSpec((1, 1, T), lambda b: (b, 0, 0)),
            pl.BlockSpec((H,), lambda b: (0,)),
            pl.BlockSpec((H,), lambda b: (0,)),
            pl.BlockSpec((24, H), lambda b: (0, 0)),
        ],
        out_specs=pl.BlockSpec((1, T, H), lambda b: (b, 0, 0)),
        out_shape=jax.ShapeDtypeStruct((B, T, H), jnp.float32),
    )(hour3, session_start, session_end, position_emb)

    node2 = node_emb.reshape(B, N, SL, LN)
    add2 = add.reshape(B, SL, LN)
    out2 = pl.pallas_call(
        _stream_body,
        in_specs=[
            pl.BlockSpec(memory_space=pltpu.VMEM),
            pl.BlockSpec(memory_space=pl.ANY),
        ],
        out_specs=pl.BlockSpec(memory_space=pl.ANY),
        out_shape=jax.ShapeDtypeStruct((B, N, SL, LN), jnp.float32),
        scratch_shapes=[
            pltpu.VMEM((Q, N_BLK, SL, LN), jnp.float32),
            pltpu.VMEM((Q, N_BLK, SL, LN), jnp.float32),
            pltpu.SemaphoreType.DMA((Q,)),
            pltpu.SemaphoreType.DMA((Q,)),
        ],
    )(add2, node2)
    return out2.reshape(B, N, T, H)
